# Initial kernel scaffold; baseline (speedup 1.0000x reference)
import jax
import jax.numpy as jnp
from jax.experimental import pallas as pl


def kernel(logit, target):
    raise NotImplementedError("write your pallas kernel here")


# speedup vs baseline: 5.7213x; 5.7213x over previous
import functools

import jax
import jax.numpy as jnp
from jax import lax
from jax.experimental import pallas as pl
from jax.experimental.pallas import tpu as pltpu
from jax.experimental.pallas import tpu_sc as plsc

N_IN = 1000000
NP = 1 << 20
ROWS, COLS = NP // 128, 128
BLK = 256
NTILES = 32
PER_TILE = NP // NTILES
CHUNK = 4096
SENTINEL = 0xFFFFFFFF


def _terms(p, y):
    return jnp.maximum(p, 0.0) - p * y + jnp.log1p(jnp.exp(-jnp.abs(p)))


def _stats_keys_body(p_ref, y_ref, keys_ref, stats_ref):
    i = pl.program_id(0)
    p = p_ref[...]
    y = y_ref[...]
    is_pos = y > 0.0
    is_neg = y == 0.0
    term = _terms(p, y)
    bu = lax.bitcast_convert_type(p, jnp.uint32)
    sign = bu >= jnp.uint32(0x80000000)
    ukey = jnp.where(sign, ~bu, bu | jnp.uint32(0x80000000))
    keys_ref[...] = jnp.where(is_neg, ukey, jnp.uint32(SENTINEL))
    npos = jnp.sum(is_pos.astype(jnp.float32))
    nneg = jnp.sum(is_neg.astype(jnp.float32))
    spos = jnp.sum(jnp.where(is_pos, term, 0.0))
    sneg = jnp.sum(jnp.where(is_neg, term, 0.0))
    mneg = jnp.max(jnp.where(is_neg, p, -jnp.inf))
    mpos = jnp.max(jnp.where(is_pos, p, -jnp.inf))
    col = lax.broadcasted_iota(jnp.int32, (1, 128), 1)
    addv = (jnp.where(col == 0, npos, 0.0) + jnp.where(col == 1, nneg, 0.0)
            + jnp.where(col == 2, spos, 0.0) + jnp.where(col == 3, sneg, 0.0))
    maxv = jnp.maximum(jnp.where(col == 4, mneg, -jnp.inf),
                       jnp.where(col == 5, mpos, -jnp.inf))

    @pl.when(i == 0)
    def _():
        stats_ref[...] = jnp.where(col >= 4, -jnp.inf, 0.0)

    stats_ref[...] = stats_ref[...] + addv
    stats_ref[...] = jnp.maximum(stats_ref[...], maxv)


def _stats_keys(p2, y2):
    return pl.pallas_call(
        _stats_keys_body,
        grid=(ROWS // BLK,),
        in_specs=[pl.BlockSpec((BLK, COLS), lambda i: (i, 0)),
                  pl.BlockSpec((BLK, COLS), lambda i: (i, 0))],
        out_specs=[pl.BlockSpec((BLK, COLS), lambda i: (i, 0)),
                   pl.BlockSpec((1, 128), lambda i: (0, 0))],
        out_shape=[jax.ShapeDtypeStruct((ROWS, COLS), jnp.uint32),
                   jax.ShapeDtypeStruct((1, 128), jnp.float32)],
    )(p2, y2)


@functools.lru_cache(maxsize=None)
def _make_hist(shift, nbuckets):
    mesh = plsc.VectorSubcoreMesh(core_axis_name="c", subcore_axis_name="s",
                                  num_cores=2, num_subcores=16)

    @functools.partial(
        pl.kernel,
        out_type=jax.ShapeDtypeStruct((NTILES, nbuckets), jnp.int32),
        mesh=mesh,
        compiler_params=pltpu.CompilerParams(needs_layout_passes=False),
        scratch_types=[
            pltpu.VMEM((CHUNK,), jnp.uint32),
            pltpu.VMEM((16,), jnp.uint32),
            pltpu.VMEM((16,), jnp.uint32),
            pltpu.VMEM((16 * nbuckets,), jnp.int32),
            pltpu.VMEM((nbuckets,), jnp.int32),
        ],
    )
    def hist(keys_hbm, lo_hbm, hi_hbm, out_hbm, keys_v, lo_v, hi_v, hist_v,
             merged_v):
        wid = lax.axis_index("s") * 2 + lax.axis_index("c")
        zero16 = jnp.zeros((16,), jnp.int32)

        def zbody(j, carry):
            hist_v[pl.ds(j * 16, 16)] = zero16
            return carry

        lax.fori_loop(0, nbuckets, zbody, 0)
        pltpu.sync_copy(lo_hbm, lo_v)
        pltpu.sync_copy(hi_hbm, hi_v)
        lo = lo_v[...]
        hi = hi_v[...]
        lane_off = lax.iota(jnp.int32, 16) * nbuckets
        ones = jnp.ones((16,), jnp.int32)
        base0 = wid * PER_TILE

        def cbody(c, carry):
            pltpu.sync_copy(keys_hbm.at[pl.ds(base0 + c * CHUNK, CHUNK)],
                            keys_v)

            def vbody(t, carry2):
                k = keys_v[pl.ds(t * 16, 16)]
                valid = (k >= lo) & (k <= hi)
                bucket = ((k - lo) >> shift).astype(jnp.int32)
                plsc.addupdate_scatter(hist_v, [lane_off + bucket], ones,
                                       mask=valid)
                return carry2

            lax.fori_loop(0, CHUNK // 16, vbody, 0)
            return carry

        lax.fori_loop(0, PER_TILE // CHUNK, cbody, 0)

        def mbody(j, carry):
            acc = hist_v[pl.ds(j * 16, 16)]
            for l in range(1, 16):
                acc = acc + hist_v[pl.ds(l * nbuckets + j * 16, 16)]
            merged_v[pl.ds(j * 16, 16)] = acc
            return carry

        lax.fori_loop(0, nbuckets // 16, mbody, 0)
        pltpu.sync_copy(merged_v, out_hbm.at[wid])

    return hist


def _sless_body(keys_ref, kstar_ref, out_ref):
    i = pl.program_id(0)
    k = keys_ref[...]
    kstar = kstar_ref[0, 0]
    sel = k < kstar
    neg = k < jnp.uint32(0x80000000)
    bu = jnp.where(neg, ~k, k ^ jnp.uint32(0x80000000))
    p = lax.bitcast_convert_type(bu, jnp.float32)
    term = jnp.maximum(p, 0.0) + jnp.log1p(jnp.exp(-jnp.abs(p)))
    part = jnp.sum(jnp.where(sel, term, 0.0), axis=0)[None, :]

    @pl.when(i == 0)
    def _():
        out_ref[...] = jnp.zeros_like(out_ref)

    out_ref[...] = out_ref[...] + part


def _sless(keys2, kstar):
    return pl.pallas_call(
        _sless_body,
        grid=(ROWS // 512,),
        in_specs=[pl.BlockSpec((512, COLS), lambda i: (i, 0)),
                  pl.BlockSpec(memory_space=pltpu.SMEM)],
        out_specs=pl.BlockSpec((1, 128), lambda i: (0, 0)),
        out_shape=jax.ShapeDtypeStruct((1, 128), jnp.float32),
    )(keys2, kstar)


def _bucket_step(hist32, rank):
    h = jnp.sum(hist32, axis=0)
    c = jnp.cumsum(h)
    b = jnp.argmax(c > rank).astype(jnp.int32)
    below = jnp.where(b > 0, c[jnp.maximum(b - 1, 0)], 0)
    return b, below


def kernel(logit, target):
    p = logit.reshape(-1)
    y = target.reshape(-1)
    npad = NP - p.shape[0]
    p2 = jnp.pad(p, (0, npad)).reshape(ROWS, COLS)
    y2 = jnp.pad(y, (0, npad), constant_values=-1.0).reshape(ROWS, COLS)
    keys2, stats = _stats_keys(p2, y2)
    s = stats[0]
    num_pos = s[0].astype(jnp.int32)
    num_neg = s[1].astype(jnp.int32)
    s_pos, s_negall, max_neg, max_pos = s[2], s[3], s[4], s[5]
    topm = jnp.minimum(num_pos, num_neg) - 1
    take = topm > 0
    r = jnp.maximum(topm - 1, 0)

    keys_flat = keys2.reshape(-1)
    full = lambda v: jnp.full((16,), v, jnp.uint32)
    h1 = _make_hist(20, 4096)(keys_flat, full(jnp.uint32(0)),
                              full(jnp.uint32(0xFFFFFFFE)))
    b1, cb1 = _bucket_step(h1, r)
    lo1 = b1.astype(jnp.uint32) << 20
    h2 = _make_hist(8, 4096)(keys_flat, full(lo1),
                             full(lo1 + jnp.uint32((1 << 20) - 1)))
    b2, cb2 = _bucket_step(h2, r - cb1)
    lo2 = lo1 + (b2.astype(jnp.uint32) << 8)
    h3 = _make_hist(0, 256)(keys_flat, full(lo2), full(lo2 + jnp.uint32(255)))
    b3, cb3 = _bucket_step(h3, r - cb1 - cb2)
    kstar = lo2 + b3.astype(jnp.uint32)
    cnt_less = cb1 + cb2 + cb3

    kstar_eff = jnp.where(take, kstar, jnp.uint32(0))
    s_less = jnp.sum(_sless(keys2, kstar_eff.reshape(1, 1)))

    bu = jnp.where(kstar >= jnp.uint32(0x80000000),
                   kstar ^ jnp.uint32(0x80000000), ~kstar)
    vstar = lax.bitcast_convert_type(bu, jnp.float32)
    tstar = jnp.maximum(vstar, 0.0) + jnp.log1p(jnp.exp(-jnp.abs(vstar)))
    s_sel = s_less + (topm - cnt_less).astype(jnp.float32) * tstar

    loss_bce = jnp.where(take, s_pos + s_sel, s_pos + s_negall)
    rank_val = jnp.maximum(0.0, 1.0 - max_neg + max_pos)
    loss_total = loss_bce + 0.1 * jnp.where(take, rank_val, 0.0)
    count = jnp.where(take, num_pos + topm, num_pos + num_neg).astype(jnp.int32)
    return loss_total, count


# speedup vs baseline: 6.9226x; 1.2100x over previous
import functools

import jax
import jax.numpy as jnp
from jax import lax
from jax.experimental import pallas as pl
from jax.experimental.pallas import tpu as pltpu
from jax.experimental.pallas import tpu_sc as plsc

N_IN = 1000000
NP = 1 << 20
ROWS, COLS = NP // 128, 128
BLK = 256
NTILES = 32
PER_TILE = NP // NTILES
CHUNK = 16384
NCHUNK = PER_TILE // CHUNK
SENTINEL = 0xFFFFFFFF


def _terms(p, y):
    return jnp.maximum(p, 0.0) - p * y + jnp.log1p(jnp.exp(-jnp.abs(p)))


def _stats_keys_body(p_ref, y_ref, keys_ref, stats_ref):
    i = pl.program_id(0)
    p = p_ref[...]
    y = y_ref[...]
    is_pos = y > 0.0
    is_neg = y == 0.0
    term = _terms(p, y)
    bu = lax.bitcast_convert_type(p, jnp.uint32)
    sign = bu >= jnp.uint32(0x80000000)
    ukey = jnp.where(sign, ~bu, bu | jnp.uint32(0x80000000))
    keys_ref[...] = jnp.where(is_neg, ukey, jnp.uint32(SENTINEL))
    npos = jnp.sum(is_pos.astype(jnp.float32))
    nneg = jnp.sum(is_neg.astype(jnp.float32))
    spos = jnp.sum(jnp.where(is_pos, term, 0.0))
    sneg = jnp.sum(jnp.where(is_neg, term, 0.0))
    mneg = jnp.max(jnp.where(is_neg, p, -jnp.inf))
    mpos = jnp.max(jnp.where(is_pos, p, -jnp.inf))
    col = lax.broadcasted_iota(jnp.int32, (1, 128), 1)
    addv = (jnp.where(col == 0, npos, 0.0) + jnp.where(col == 1, nneg, 0.0)
            + jnp.where(col == 2, spos, 0.0) + jnp.where(col == 3, sneg, 0.0))
    maxv = jnp.maximum(jnp.where(col == 4, mneg, -jnp.inf),
                       jnp.where(col == 5, mpos, -jnp.inf))

    @pl.when(i == 0)
    def _():
        stats_ref[...] = jnp.where(col >= 4, -jnp.inf, 0.0)

    stats_ref[...] = stats_ref[...] + addv
    stats_ref[...] = jnp.maximum(stats_ref[...], maxv)


def _stats_keys(p2, y2):
    return pl.pallas_call(
        _stats_keys_body,
        grid=(ROWS // BLK,),
        in_specs=[pl.BlockSpec((BLK, COLS), lambda i: (i, 0)),
                  pl.BlockSpec((BLK, COLS), lambda i: (i, 0))],
        out_specs=[pl.BlockSpec((BLK, COLS), lambda i: (i, 0)),
                   pl.BlockSpec((1, 128), lambda i: (0, 0))],
        out_shape=[jax.ShapeDtypeStruct((ROWS, COLS), jnp.uint32),
                   jax.ShapeDtypeStruct((1, 128), jnp.float32)],
    )(p2, y2)


@functools.lru_cache(maxsize=None)
def _make_hist(shift, nbuckets, masked):
    mesh = plsc.VectorSubcoreMesh(core_axis_name="c", subcore_axis_name="s",
                                  num_cores=2, num_subcores=16)
    nin = 3 if masked else 1
    scratch = [
        pltpu.VMEM((CHUNK,), jnp.uint32),
        pltpu.VMEM((CHUNK,), jnp.uint32),
        pltpu.VMEM((16,), jnp.uint32),
        pltpu.VMEM((16,), jnp.uint32),
        pltpu.VMEM((16 * nbuckets,), jnp.int32),
        pltpu.VMEM((nbuckets,), jnp.int32),
        pltpu.SemaphoreType.DMA,
        pltpu.SemaphoreType.DMA,
    ]

    @functools.partial(
        pl.kernel,
        out_type=jax.ShapeDtypeStruct((NTILES, nbuckets), jnp.int32),
        mesh=mesh,
        compiler_params=pltpu.CompilerParams(needs_layout_passes=False),
        scratch_types=scratch,
    )
    def hist(*refs):
        args, rest = refs[:nin], refs[nin:]
        keys_hbm = args[0]
        out_hbm = rest[0]
        (buf0, buf1, lo_v, hi_v, hist_v, merged_v, sem0, sem1) = rest[1:]
        bufs, sems = (buf0, buf1), (sem0, sem1)
        wid = lax.axis_index("s") * 2 + lax.axis_index("c")
        base0 = wid * PER_TILE
        cps = [None, None]
        cps[0] = pltpu.async_copy(keys_hbm.at[pl.ds(base0, CHUNK)], buf0,
                                  sem0)
        if masked:
            pltpu.sync_copy(args[1], lo_v)
            pltpu.sync_copy(args[2], hi_v)
            lo = lo_v[...]
            hi = hi_v[...]
        zero16 = jnp.zeros((16,), jnp.int32)

        def zbody(j, carry):
            for u in range(16):
                hist_v[pl.ds((j * 16 + u) * 16, 16)] = zero16
            return carry

        lax.fori_loop(0, nbuckets // 16, zbody, 0)
        lane_off = lax.iota(jnp.int32, 16) * nbuckets
        ones = jnp.ones((16,), jnp.int32)

        for c in range(NCHUNK):
            cps[c % 2].wait()
            if c + 1 < NCHUNK:
                cps[(c + 1) % 2] = pltpu.async_copy(
                    keys_hbm.at[pl.ds(base0 + (c + 1) * CHUNK, CHUNK)],
                    bufs[(c + 1) % 2], sems[(c + 1) % 2])
            buf = bufs[c % 2]

            def vbody(t, carry2):
                for u in range(16):
                    k = buf[pl.ds((t * 16 + u) * 16, 16)]
                    if masked:
                        valid = (k >= lo) & (k <= hi)
                        bucket = ((k - lo) >> shift).astype(jnp.int32)
                        plsc.addupdate_scatter(hist_v, [lane_off + bucket],
                                               ones, mask=valid)
                    else:
                        bucket = (k >> shift).astype(jnp.int32)
                        plsc.addupdate_scatter(hist_v, [lane_off + bucket],
                                               ones)
                return carry2

            lax.fori_loop(0, CHUNK // 256, vbody, 0)

        def mbody(j, carry):
            for u in range(2):
                jj = j * 2 + u
                acc = hist_v[pl.ds(jj * 16, 16)]
                for l in range(1, 16):
                    acc = acc + hist_v[pl.ds(l * nbuckets + jj * 16, 16)]
                merged_v[pl.ds(jj * 16, 16)] = acc
            return carry

        lax.fori_loop(0, nbuckets // 32, mbody, 0)
        pltpu.sync_copy(merged_v, out_hbm.at[wid])

    return hist


def _sless_body(keys_ref, kstar_ref, out_ref):
    i = pl.program_id(0)
    k = keys_ref[...]
    kstar = kstar_ref[0, 0]
    sel = k < kstar
    neg = k < jnp.uint32(0x80000000)
    bu = jnp.where(neg, ~k, k ^ jnp.uint32(0x80000000))
    p = lax.bitcast_convert_type(bu, jnp.float32)
    term = jnp.maximum(p, 0.0) + jnp.log1p(jnp.exp(-jnp.abs(p)))
    part = jnp.sum(jnp.where(sel, term, 0.0), axis=0)[None, :]

    @pl.when(i == 0)
    def _():
        out_ref[...] = jnp.zeros_like(out_ref)

    out_ref[...] = out_ref[...] + part


def _sless(keys2, kstar):
    return pl.pallas_call(
        _sless_body,
        grid=(ROWS // 512,),
        in_specs=[pl.BlockSpec((512, COLS), lambda i: (i, 0)),
                  pl.BlockSpec(memory_space=pltpu.SMEM)],
        out_specs=pl.BlockSpec((1, 128), lambda i: (0, 0)),
        out_shape=jax.ShapeDtypeStruct((1, 128), jnp.float32),
    )(keys2, kstar)


def _bucket_step(hist32, rank):
    h = jnp.sum(hist32, axis=0)
    c = jnp.cumsum(h)
    b = jnp.argmax(c > rank).astype(jnp.int32)
    below = jnp.where(b > 0, c[jnp.maximum(b - 1, 0)], 0)
    return b, below


def kernel(logit, target):
    p = logit.reshape(-1)
    y = target.reshape(-1)
    npad = NP - p.shape[0]
    p2 = jnp.pad(p, (0, npad)).reshape(ROWS, COLS)
    y2 = jnp.pad(y, (0, npad), constant_values=-1.0).reshape(ROWS, COLS)
    keys2, stats = _stats_keys(p2, y2)
    s = stats[0]
    num_pos = s[0].astype(jnp.int32)
    num_neg = s[1].astype(jnp.int32)
    s_pos, s_negall, max_neg, max_pos = s[2], s[3], s[4], s[5]
    topm = jnp.minimum(num_pos, num_neg) - 1
    take = topm > 0
    r = jnp.maximum(topm - 1, 0)

    keys_flat = keys2.reshape(-1)
    full = lambda v: jnp.full((16,), v, jnp.uint32)
    h1 = _make_hist(21, 2048, False)(keys_flat)
    b1, cb1 = _bucket_step(h1, r)
    lo1 = b1.astype(jnp.uint32) << 21
    h2 = _make_hist(10, 2048, True)(keys_flat, full(lo1),
                                    full(lo1 + jnp.uint32((1 << 21) - 1)))
    b2, cb2 = _bucket_step(h2, r - cb1)
    lo2 = lo1 + (b2.astype(jnp.uint32) << 10)
    h3 = _make_hist(0, 1024, True)(keys_flat, full(lo2),
                                   full(lo2 + jnp.uint32(1023)))
    b3, cb3 = _bucket_step(h3, r - cb1 - cb2)
    kstar = lo2 + b3.astype(jnp.uint32)
    cnt_less = cb1 + cb2 + cb3

    kstar_eff = jnp.where(take, kstar, jnp.uint32(0))
    s_less = jnp.sum(_sless(keys2, kstar_eff.reshape(1, 1)))

    bu = jnp.where(kstar >= jnp.uint32(0x80000000),
                   kstar ^ jnp.uint32(0x80000000), ~kstar)
    vstar = lax.bitcast_convert_type(bu, jnp.float32)
    tstar = jnp.maximum(vstar, 0.0) + jnp.log1p(jnp.exp(-jnp.abs(vstar)))
    s_sel = s_less + (topm - cnt_less).astype(jnp.float32) * tstar

    loss_bce = jnp.where(take, s_pos + s_sel, s_pos + s_negall)
    rank_val = jnp.maximum(0.0, 1.0 - max_neg + max_pos)
    loss_total = loss_bce + 0.1 * jnp.where(take, rank_val, 0.0)
    count = jnp.where(take, num_pos + topm, num_pos + num_neg).astype(jnp.int32)
    return loss_total, count


# speedup vs baseline: 9.8590x; 1.4242x over previous
import functools

import jax
import jax.numpy as jnp
from jax import lax
from jax.experimental import pallas as pl
from jax.experimental.pallas import tpu as pltpu
from jax.experimental.pallas import tpu_sc as plsc

N_IN = 1000000
NP = 1 << 20
ROWS, COLS = NP // 128, 128
BLK = 256
NTILES = 32
PER_TILE = NP // NTILES
CHUNK = 16384
NCHUNK = PER_TILE // CHUNK
SENTINEL = 0xFFFFFFFF


def _terms(p, y):
    return jnp.maximum(p, 0.0) - p * y + jnp.log1p(jnp.exp(-jnp.abs(p)))


def _stats_keys_body(p_ref, y_ref, keys_ref, stats_ref):
    i = pl.program_id(0)
    p = p_ref[...]
    y = y_ref[...]
    is_pos = y > 0.0
    is_neg = y == 0.0
    term = _terms(p, y)
    bu = lax.bitcast_convert_type(p, jnp.uint32)
    sign = bu >= jnp.uint32(0x80000000)
    ukey = jnp.where(sign, ~bu, bu | jnp.uint32(0x80000000))
    keys_ref[...] = jnp.where(is_neg, ukey, jnp.uint32(SENTINEL))
    npos = jnp.sum(is_pos.astype(jnp.float32))
    nneg = jnp.sum(is_neg.astype(jnp.float32))
    spos = jnp.sum(jnp.where(is_pos, term, 0.0))
    sneg = jnp.sum(jnp.where(is_neg, term, 0.0))
    mneg = jnp.max(jnp.where(is_neg, p, -jnp.inf))
    mpos = jnp.max(jnp.where(is_pos, p, -jnp.inf))
    col = lax.broadcasted_iota(jnp.int32, (1, 128), 1)
    addv = (jnp.where(col == 0, npos, 0.0) + jnp.where(col == 1, nneg, 0.0)
            + jnp.where(col == 2, spos, 0.0) + jnp.where(col == 3, sneg, 0.0))
    maxv = jnp.maximum(jnp.where(col == 4, mneg, -jnp.inf),
                       jnp.where(col == 5, mpos, -jnp.inf))

    @pl.when(i == 0)
    def _():
        stats_ref[...] = jnp.where(col >= 4, -jnp.inf, 0.0)

    stats_ref[...] = stats_ref[...] + addv
    stats_ref[...] = jnp.maximum(stats_ref[...], maxv)


def _stats_keys(p2, y2):
    return pl.pallas_call(
        _stats_keys_body,
        grid=(ROWS // BLK,),
        in_specs=[pl.BlockSpec((BLK, COLS), lambda i: (i, 0)),
                  pl.BlockSpec((BLK, COLS), lambda i: (i, 0))],
        out_specs=[pl.BlockSpec((BLK, COLS), lambda i: (i, 0)),
                   pl.BlockSpec((1, 128), lambda i: (0, 0))],
        out_shape=[jax.ShapeDtypeStruct((ROWS, COLS), jnp.uint32),
                   jax.ShapeDtypeStruct((1, 128), jnp.float32)],
    )(p2, y2)


@functools.lru_cache(maxsize=None)
def _make_hist(shift, nbuckets, masked):
    mesh = plsc.VectorSubcoreMesh(core_axis_name="c", subcore_axis_name="s",
                                  num_cores=2, num_subcores=16)
    nin = 3 if masked else 1
    scratch = [
        pltpu.VMEM((CHUNK,), jnp.uint32),
        pltpu.VMEM((CHUNK,), jnp.uint32),
        pltpu.VMEM((16,), jnp.uint32),
        pltpu.VMEM((16,), jnp.uint32),
        pltpu.VMEM((16 * nbuckets,), jnp.int32),
        pltpu.VMEM((nbuckets,), jnp.int32),
        pltpu.SemaphoreType.DMA,
        pltpu.SemaphoreType.DMA,
    ]

    @functools.partial(
        pl.kernel,
        out_type=jax.ShapeDtypeStruct((NTILES, nbuckets), jnp.int32),
        mesh=mesh,
        compiler_params=pltpu.CompilerParams(needs_layout_passes=False),
        scratch_types=scratch,
    )
    def hist(*refs):
        args, rest = refs[:nin], refs[nin:]
        keys_hbm = args[0]
        out_hbm = rest[0]
        (buf0, buf1, lo_v, hi_v, hist_v, merged_v, sem0, sem1) = rest[1:]
        bufs, sems = (buf0, buf1), (sem0, sem1)
        wid = lax.axis_index("s") * 2 + lax.axis_index("c")
        base0 = wid * PER_TILE
        cps = [None, None]
        cps[0] = pltpu.async_copy(keys_hbm.at[pl.ds(base0, CHUNK)], buf0,
                                  sem0)
        if masked:
            pltpu.sync_copy(args[1], lo_v)
            pltpu.sync_copy(args[2], hi_v)
            lo = lo_v[...]
            hi = hi_v[...]
        zero16 = jnp.zeros((16,), jnp.int32)

        def zbody(j, carry):
            for u in range(16):
                hist_v[pl.ds((j * 16 + u) * 16, 16)] = zero16
            return carry

        lax.fori_loop(0, nbuckets // 16, zbody, 0)
        lane_off = lax.iota(jnp.int32, 16) * nbuckets
        ones = jnp.ones((16,), jnp.int32)

        for c in range(NCHUNK):
            cps[c % 2].wait()
            if c + 1 < NCHUNK:
                cps[(c + 1) % 2] = pltpu.async_copy(
                    keys_hbm.at[pl.ds(base0 + (c + 1) * CHUNK, CHUNK)],
                    bufs[(c + 1) % 2], sems[(c + 1) % 2])
            buf = bufs[c % 2]

            def vbody(t, carry2):
                for u in range(16):
                    k = buf[pl.ds((t * 16 + u) * 16, 16)]
                    if masked:
                        valid = (k >= lo) & (k <= hi)
                        bucket = ((k - lo) >> shift).astype(jnp.int32)
                        plsc.addupdate_scatter(hist_v, [lane_off + bucket],
                                               ones, mask=valid)
                    else:
                        bucket = (k >> shift).astype(jnp.int32)
                        plsc.addupdate_scatter(hist_v, [lane_off + bucket],
                                               ones)
                return carry2

            lax.fori_loop(0, CHUNK // 256, vbody, 0)

        def mbody(j, carry):
            for u in range(2):
                jj = j * 2 + u
                acc = hist_v[pl.ds(jj * 16, 16)]
                for l in range(1, 16):
                    acc = acc + hist_v[pl.ds(l * nbuckets + jj * 16, 16)]
                merged_v[pl.ds(jj * 16, 16)] = acc
            return carry

        lax.fori_loop(0, nbuckets // 32, mbody, 0)
        pltpu.sync_copy(merged_v, out_hbm.at[wid])

    return hist


def _sless_body(keys_ref, kstar_ref, out_ref):
    i = pl.program_id(0)
    k = keys_ref[...]
    kstar = kstar_ref[0, 0]
    sel = k < kstar
    neg = k < jnp.uint32(0x80000000)
    bu = jnp.where(neg, ~k, k ^ jnp.uint32(0x80000000))
    p = lax.bitcast_convert_type(bu, jnp.float32)
    term = jnp.maximum(p, 0.0) + jnp.log1p(jnp.exp(-jnp.abs(p)))
    part = jnp.sum(jnp.where(sel, term, 0.0), axis=0)[None, :]

    @pl.when(i == 0)
    def _():
        out_ref[...] = jnp.zeros_like(out_ref)

    out_ref[...] = out_ref[...] + part


def _sless(keys2, kstar):
    return pl.pallas_call(
        _sless_body,
        grid=(ROWS // 512,),
        in_specs=[pl.BlockSpec((512, COLS), lambda i: (i, 0)),
                  pl.BlockSpec(memory_space=pltpu.SMEM)],
        out_specs=pl.BlockSpec((1, 128), lambda i: (0, 0)),
        out_shape=jax.ShapeDtypeStruct((1, 128), jnp.float32),
    )(keys2, kstar)


def _bucket_step(hist32, rank):
    h = jnp.sum(hist32, axis=0)
    c = jnp.cumsum(h)
    b = jnp.argmax(c > rank).astype(jnp.int32)
    below = jnp.where(b > 0, c[jnp.maximum(b - 1, 0)], 0)
    return b, below


def kernel(logit, target):
    npad = NP - logit.shape[0]
    p2 = jnp.pad(logit, ((0, npad), (0, 0))).reshape(ROWS, COLS)
    y2 = jnp.pad(target, ((0, npad), (0, 0)),
                 constant_values=-1.0).reshape(ROWS, COLS)
    keys2, stats = _stats_keys(p2, y2)
    s = stats[0]
    num_pos = s[0].astype(jnp.int32)
    num_neg = s[1].astype(jnp.int32)
    s_pos, s_negall, max_neg, max_pos = s[2], s[3], s[4], s[5]
    topm = jnp.minimum(num_pos, num_neg) - 1
    take = topm > 0
    r = jnp.maximum(topm - 1, 0)

    keys_flat = keys2.reshape(-1)
    full = lambda v: jnp.full((16,), v, jnp.uint32)
    h1 = _make_hist(21, 2048, False)(keys_flat)
    b1, cb1 = _bucket_step(h1, r)
    lo1 = b1.astype(jnp.uint32) << 21
    h2 = _make_hist(10, 2048, True)(keys_flat, full(lo1),
                                    full(lo1 + jnp.uint32((1 << 21) - 1)))
    b2, cb2 = _bucket_step(h2, r - cb1)
    lo2 = lo1 + (b2.astype(jnp.uint32) << 10)
    h3 = _make_hist(0, 1024, True)(keys_flat, full(lo2),
                                   full(lo2 + jnp.uint32(1023)))
    b3, cb3 = _bucket_step(h3, r - cb1 - cb2)
    kstar = lo2 + b3.astype(jnp.uint32)
    cnt_less = cb1 + cb2 + cb3

    kstar_eff = jnp.where(take, kstar, jnp.uint32(0))
    s_less = jnp.sum(_sless(keys2, kstar_eff.reshape(1, 1)))

    bu = jnp.where(kstar >= jnp.uint32(0x80000000),
                   kstar ^ jnp.uint32(0x80000000), ~kstar)
    vstar = lax.bitcast_convert_type(bu, jnp.float32)
    tstar = jnp.maximum(vstar, 0.0) + jnp.log1p(jnp.exp(-jnp.abs(vstar)))
    s_sel = s_less + (topm - cnt_less).astype(jnp.float32) * tstar

    loss_bce = jnp.where(take, s_pos + s_sel, s_pos + s_negall)
    rank_val = jnp.maximum(0.0, 1.0 - max_neg + max_pos)
    loss_total = loss_bce + 0.1 * jnp.where(take, rank_val, 0.0)
    count = jnp.where(take, num_pos + topm, num_pos + num_neg).astype(jnp.int32)
    return loss_total, count


# speedup vs baseline: 11.7247x; 1.1892x over previous
import functools

import jax
import jax.numpy as jnp
from jax import lax
from jax.experimental import pallas as pl
from jax.experimental.pallas import tpu as pltpu
from jax.experimental.pallas import tpu_sc as plsc

N_IN = 1000000
NP = 1 << 20
ROWS, COLS = NP // 128, 128
BLK = 256
NTILES = 32
PER_TILE = NP // NTILES
CHUNK = 16384
NCHUNK = PER_TILE // CHUNK
SENTINEL = 0xFFFFFFFF


def _terms(p, y):
    return jnp.maximum(p, 0.0) - p * y + jnp.log1p(jnp.exp(-jnp.abs(p)))


def _stats_keys_body(p_ref, y_ref, keys_ref, stats_ref):
    i = pl.program_id(0)
    p = p_ref[...]
    y = y_ref[...]
    is_pos = y > 0.0
    is_neg = y == 0.0
    term = _terms(p, y)
    bu = lax.bitcast_convert_type(p, jnp.uint32)
    sign = bu >= jnp.uint32(0x80000000)
    ukey = jnp.where(sign, ~bu, bu | jnp.uint32(0x80000000))
    keys_ref[...] = jnp.where(is_neg, ukey, jnp.uint32(SENTINEL))
    npos = jnp.sum(is_pos.astype(jnp.float32))
    nneg = jnp.sum(is_neg.astype(jnp.float32))
    spos = jnp.sum(jnp.where(is_pos, term, 0.0))
    sneg = jnp.sum(jnp.where(is_neg, term, 0.0))
    mneg = jnp.max(jnp.where(is_neg, p, -jnp.inf))
    mpos = jnp.max(jnp.where(is_pos, p, -jnp.inf))
    col = lax.broadcasted_iota(jnp.int32, (1, 128), 1)
    addv = (jnp.where(col == 0, npos, 0.0) + jnp.where(col == 1, nneg, 0.0)
            + jnp.where(col == 2, spos, 0.0) + jnp.where(col == 3, sneg, 0.0))
    maxv = jnp.maximum(jnp.where(col == 4, mneg, -jnp.inf),
                       jnp.where(col == 5, mpos, -jnp.inf))

    @pl.when(i == 0)
    def _():
        stats_ref[...] = jnp.where(col >= 4, -jnp.inf, 0.0)

    stats_ref[...] = stats_ref[...] + addv
    stats_ref[...] = jnp.maximum(stats_ref[...], maxv)


def _stats_keys(p2, y2):
    return pl.pallas_call(
        _stats_keys_body,
        grid=(ROWS // BLK,),
        in_specs=[pl.BlockSpec((BLK, COLS), lambda i: (i, 0)),
                  pl.BlockSpec((BLK, COLS), lambda i: (i, 0))],
        out_specs=[pl.BlockSpec((BLK, COLS), lambda i: (i, 0)),
                   pl.BlockSpec((1, 128), lambda i: (0, 0))],
        out_shape=[jax.ShapeDtypeStruct((ROWS, COLS), jnp.uint32),
                   jax.ShapeDtypeStruct((1, 128), jnp.float32)],
    )(p2, y2)


@functools.lru_cache(maxsize=None)
def _make_hist(shift, nbuckets, masked):
    mesh = plsc.VectorSubcoreMesh(core_axis_name="c", subcore_axis_name="s",
                                  num_cores=2, num_subcores=16)
    nin = 3 if masked else 1
    scratch = [
        pltpu.VMEM((CHUNK,), jnp.uint32),
        pltpu.VMEM((CHUNK,), jnp.uint32),
        pltpu.VMEM((16,), jnp.uint32),
        pltpu.VMEM((16,), jnp.uint32),
        pltpu.VMEM((16 * nbuckets + 16,), jnp.int32),
        pltpu.VMEM((nbuckets,), jnp.int32),
        pltpu.SemaphoreType.DMA,
        pltpu.SemaphoreType.DMA,
    ]

    @functools.partial(
        pl.kernel,
        out_type=jax.ShapeDtypeStruct((NTILES, nbuckets), jnp.int32),
        mesh=mesh,
        compiler_params=pltpu.CompilerParams(needs_layout_passes=False),
        scratch_types=scratch,
    )
    def hist(*refs):
        args, rest = refs[:nin], refs[nin:]
        keys_hbm = args[0]
        out_hbm = rest[0]
        (buf0, buf1, lo_v, hi_v, hist_v, merged_v, sem0, sem1) = rest[1:]
        bufs, sems = (buf0, buf1), (sem0, sem1)
        wid = lax.axis_index("s") * 2 + lax.axis_index("c")
        base0 = wid * PER_TILE
        cps = [None, None]
        cps[0] = pltpu.async_copy(keys_hbm.at[pl.ds(base0, CHUNK)], buf0,
                                  sem0)
        if masked:
            pltpu.sync_copy(args[1], lo_v)
            pltpu.sync_copy(args[2], hi_v)
            lo = lo_v[...]
            hi = hi_v[...]
        zero16 = jnp.zeros((16,), jnp.int32)

        def zbody(j, carry):
            for u in range(16):
                hist_v[pl.ds((j * 16 + u) * 16, 16)] = zero16
            return carry

        lax.fori_loop(0, nbuckets // 16, zbody, 0)
        hist_v[pl.ds(nbuckets * 16, 16)] = zero16
        lane_off = lax.iota(jnp.int32, 16) * (nbuckets + 1)
        ones = jnp.ones((16,), jnp.int32)

        for c in range(NCHUNK):
            cps[c % 2].wait()
            if c + 1 < NCHUNK:
                cps[(c + 1) % 2] = pltpu.async_copy(
                    keys_hbm.at[pl.ds(base0 + (c + 1) * CHUNK, CHUNK)],
                    bufs[(c + 1) % 2], sems[(c + 1) % 2])
            buf = bufs[c % 2]

            def vbody(t, carry2):
                for u in range(16):
                    k = buf[pl.ds((t * 16 + u) * 16, 16)]
                    if masked:
                        valid = (k >= lo) & (k <= hi)
                        bucket = ((k - lo) >> shift).astype(jnp.int32)
                        plsc.addupdate_scatter(hist_v, [lane_off + bucket],
                                               ones, mask=valid)
                    else:
                        bucket = (k >> shift).astype(jnp.int32)
                        plsc.addupdate_scatter(hist_v, [lane_off + bucket],
                                               ones)
                return carry2

            lax.fori_loop(0, CHUNK // 256, vbody, 0)

        def mbody(j, carry):
            for u in range(2):
                jj = j * 2 + u
                acc = hist_v[pl.ds(jj * 16, 16)]
                for l in range(1, 16):
                    acc = acc + hist_v[pl.ds(l * (nbuckets + 1) + jj * 16,
                                             16)]
                merged_v[pl.ds(jj * 16, 16)] = acc
            return carry

        lax.fori_loop(0, nbuckets // 32, mbody, 0)
        pltpu.sync_copy(merged_v, out_hbm.at[wid])

    return hist


def _sless_body(keys_ref, kstar_ref, out_ref):
    i = pl.program_id(0)
    k = keys_ref[...]
    kstar = kstar_ref[0, 0]
    sel = k < kstar
    neg = k < jnp.uint32(0x80000000)
    bu = jnp.where(neg, ~k, k ^ jnp.uint32(0x80000000))
    p = lax.bitcast_convert_type(bu, jnp.float32)
    term = jnp.maximum(p, 0.0) + jnp.log1p(jnp.exp(-jnp.abs(p)))
    part = jnp.sum(jnp.where(sel, term, 0.0), axis=0)[None, :]

    @pl.when(i == 0)
    def _():
        out_ref[...] = jnp.zeros_like(out_ref)

    out_ref[...] = out_ref[...] + part


def _sless(keys2, kstar):
    return pl.pallas_call(
        _sless_body,
        grid=(ROWS // 512,),
        in_specs=[pl.BlockSpec((512, COLS), lambda i: (i, 0)),
                  pl.BlockSpec(memory_space=pltpu.SMEM)],
        out_specs=pl.BlockSpec((1, 128), lambda i: (0, 0)),
        out_shape=jax.ShapeDtypeStruct((1, 128), jnp.float32),
    )(keys2, kstar)


def _bucket_step(hist32, rank):
    h = jnp.sum(hist32, axis=0)
    c = jnp.cumsum(h)
    b = jnp.argmax(c > rank).astype(jnp.int32)
    below = jnp.where(b > 0, c[jnp.maximum(b - 1, 0)], 0)
    return b, below


def kernel(logit, target):
    npad = NP - logit.shape[0]
    p2 = jnp.pad(logit, ((0, npad), (0, 0))).reshape(ROWS, COLS)
    y2 = jnp.pad(target, ((0, npad), (0, 0)),
                 constant_values=-1.0).reshape(ROWS, COLS)
    keys2, stats = _stats_keys(p2, y2)
    s = stats[0]
    num_pos = s[0].astype(jnp.int32)
    num_neg = s[1].astype(jnp.int32)
    s_pos, s_negall, max_neg, max_pos = s[2], s[3], s[4], s[5]
    topm = jnp.minimum(num_pos, num_neg) - 1
    take = topm > 0
    r = jnp.maximum(topm - 1, 0)

    keys_flat = keys2.reshape(-1)
    full = lambda v: jnp.full((16,), v, jnp.uint32)
    h1 = _make_hist(21, 2048, True)(keys_flat, full(jnp.uint32(0)),
                                    full(jnp.uint32(0xFFFFFFFE)))
    b1, cb1 = _bucket_step(h1, r)
    lo1 = b1.astype(jnp.uint32) << 21
    h2 = _make_hist(10, 2048, True)(keys_flat, full(lo1),
                                    full(lo1 + jnp.uint32((1 << 21) - 1)))
    b2, cb2 = _bucket_step(h2, r - cb1)
    lo2 = lo1 + (b2.astype(jnp.uint32) << 10)
    h3 = _make_hist(0, 1024, True)(keys_flat, full(lo2),
                                   full(lo2 + jnp.uint32(1023)))
    lo2_eff = jnp.where(take, lo2, jnp.uint32(0))
    s12 = jnp.sum(_sless(keys2, lo2_eff.reshape(1, 1)))

    htot3 = jnp.sum(h3, axis=0)
    c3 = jnp.cumsum(htot3)
    b3 = jnp.argmax(c3 > (r - cb1 - cb2)).astype(jnp.int32)
    cb3 = jnp.where(b3 > 0, c3[jnp.maximum(b3 - 1, 0)], 0)
    kstar = lo2 + b3.astype(jnp.uint32)
    cnt_less = cb1 + cb2 + cb3

    jidx = jnp.arange(1024, dtype=jnp.uint32)
    kj = lo2 + jidx
    buj = jnp.where(kj >= jnp.uint32(0x80000000),
                    kj ^ jnp.uint32(0x80000000), ~kj)
    vj = lax.bitcast_convert_type(buj, jnp.float32)
    tj = jnp.maximum(vj, 0.0) + jnp.log1p(jnp.exp(-jnp.abs(vj)))
    selj = (jidx < b3.astype(jnp.uint32)) & (htot3 > 0)
    s3 = jnp.sum(jnp.where(selj, htot3.astype(jnp.float32) * tj, 0.0))
    s_less = s12 + s3

    bu = jnp.where(kstar >= jnp.uint32(0x80000000),
                   kstar ^ jnp.uint32(0x80000000), ~kstar)
    vstar = lax.bitcast_convert_type(bu, jnp.float32)
    tstar = jnp.maximum(vstar, 0.0) + jnp.log1p(jnp.exp(-jnp.abs(vstar)))
    s_sel = s_less + (topm - cnt_less).astype(jnp.float32) * tstar

    loss_bce = jnp.where(take, s_pos + s_sel, s_pos + s_negall)
    rank_val = jnp.maximum(0.0, 1.0 - max_neg + max_pos)
    loss_total = loss_bce + 0.1 * jnp.where(take, rank_val, 0.0)
    count = jnp.where(take, num_pos + topm, num_pos + num_neg).astype(jnp.int32)
    return loss_total, count


# speedup vs baseline: 11.9789x; 1.0217x over previous
import functools

import jax
import jax.numpy as jnp
from jax import lax
from jax.experimental import pallas as pl
from jax.experimental.pallas import tpu as pltpu
from jax.experimental.pallas import tpu_sc as plsc

N_IN = 1000000
NP = 1 << 20
ROWS, COLS = NP // 128, 128
BLK = 256
NTILES = 32
PER_TILE = NP // NTILES
CHUNK = 16384
NCHUNK = PER_TILE // CHUNK
SENTINEL = 0xFFFFFFFF


def _terms(p, y):
    return jnp.maximum(p, 0.0) - p * y + jnp.log1p(jnp.exp(-jnp.abs(p)))


def _stats_keys_body(p_ref, y_ref, keys_ref, stats_ref):
    i = pl.program_id(0)
    p = p_ref[...]
    y = y_ref[...]
    is_pos = y > 0.0
    is_neg = y == 0.0
    term = _terms(p, y)
    bu = lax.bitcast_convert_type(p, jnp.uint32)
    sign = bu >= jnp.uint32(0x80000000)
    ukey = jnp.where(sign, ~bu, bu | jnp.uint32(0x80000000))
    keys_ref[...] = jnp.where(is_neg, ukey, jnp.uint32(SENTINEL))
    npos = jnp.sum(is_pos.astype(jnp.float32))
    nneg = jnp.sum(is_neg.astype(jnp.float32))
    spos = jnp.sum(jnp.where(is_pos, term, 0.0))
    sneg = jnp.sum(jnp.where(is_neg, term, 0.0))
    mneg = jnp.max(jnp.where(is_neg, p, -jnp.inf))
    mpos = jnp.max(jnp.where(is_pos, p, -jnp.inf))
    col = lax.broadcasted_iota(jnp.int32, (1, 128), 1)
    addv = (jnp.where(col == 0, npos, 0.0) + jnp.where(col == 1, nneg, 0.0)
            + jnp.where(col == 2, spos, 0.0) + jnp.where(col == 3, sneg, 0.0))
    maxv = jnp.maximum(jnp.where(col == 4, mneg, -jnp.inf),
                       jnp.where(col == 5, mpos, -jnp.inf))

    @pl.when(i == 0)
    def _():
        stats_ref[...] = jnp.where(col >= 4, -jnp.inf, 0.0)

    stats_ref[...] = stats_ref[...] + addv
    stats_ref[...] = jnp.maximum(stats_ref[...], maxv)


def _stats_keys(p2, y2):
    return pl.pallas_call(
        _stats_keys_body,
        grid=(ROWS // BLK,),
        in_specs=[pl.BlockSpec((BLK, COLS), lambda i: (i, 0)),
                  pl.BlockSpec((BLK, COLS), lambda i: (i, 0))],
        out_specs=[pl.BlockSpec((BLK, COLS), lambda i: (i, 0)),
                   pl.BlockSpec((1, 128), lambda i: (0, 0))],
        out_shape=[jax.ShapeDtypeStruct((ROWS, COLS), jnp.uint32),
                   jax.ShapeDtypeStruct((1, 128), jnp.float32)],
    )(p2, y2)


@functools.lru_cache(maxsize=None)
def _make_hist(shift, nbuckets, span):
    mesh = plsc.VectorSubcoreMesh(core_axis_name="c", subcore_axis_name="s",
                                  num_cores=2, num_subcores=16)
    nin = 2
    scratch = [
        pltpu.VMEM((CHUNK,), jnp.uint32),
        pltpu.VMEM((CHUNK,), jnp.uint32),
        pltpu.VMEM((16,), jnp.uint32),
        pltpu.VMEM((16 * nbuckets + 16,), jnp.int32),
        pltpu.VMEM((nbuckets,), jnp.int32),
        pltpu.SemaphoreType.DMA,
        pltpu.SemaphoreType.DMA,
    ]

    @functools.partial(
        pl.kernel,
        out_type=jax.ShapeDtypeStruct((NTILES, nbuckets), jnp.int32),
        mesh=mesh,
        compiler_params=pltpu.CompilerParams(needs_layout_passes=False),
        scratch_types=scratch,
    )
    def hist(*refs):
        args, rest = refs[:nin], refs[nin:]
        keys_hbm = args[0]
        out_hbm = rest[0]
        (buf0, buf1, lo_v, hist_v, merged_v, sem0, sem1) = rest[1:]
        bufs, sems = (buf0, buf1), (sem0, sem1)
        wid = lax.axis_index("s") * 2 + lax.axis_index("c")
        base0 = wid * PER_TILE
        cps = [None, None]
        cps[0] = pltpu.async_copy(keys_hbm.at[pl.ds(base0, CHUNK)], buf0,
                                  sem0)
        pltpu.sync_copy(args[1], lo_v)
        lo = lo_v[...]
        span_u = jnp.uint32(span)
        zero16 = jnp.zeros((16,), jnp.int32)

        def zbody(j, carry):
            for u in range(16):
                hist_v[pl.ds((j * 16 + u) * 16, 16)] = zero16
            return carry

        lax.fori_loop(0, nbuckets // 16, zbody, 0)
        hist_v[pl.ds(nbuckets * 16, 16)] = zero16
        lane_off = lax.iota(jnp.int32, 16) * (nbuckets + 1)
        ones = jnp.ones((16,), jnp.int32)

        for c in range(NCHUNK):
            cps[c % 2].wait()
            if c + 1 < NCHUNK:
                cps[(c + 1) % 2] = pltpu.async_copy(
                    keys_hbm.at[pl.ds(base0 + (c + 1) * CHUNK, CHUNK)],
                    bufs[(c + 1) % 2], sems[(c + 1) % 2])
            buf = bufs[c % 2]

            def vbody(t, carry2):
                for u in range(16):
                    k = buf[pl.ds((t * 16 + u) * 16, 16)]
                    d = k - lo
                    valid = d <= span_u
                    bucket = (d >> shift).astype(jnp.int32)
                    plsc.addupdate_scatter(hist_v, [lane_off + bucket],
                                           ones, mask=valid)
                return carry2

            lax.fori_loop(0, CHUNK // 256, vbody, 0)

        def mbody(j, carry):
            for u in range(2):
                jj = j * 2 + u
                acc = hist_v[pl.ds(jj * 16, 16)]
                for l in range(1, 16):
                    acc = acc + hist_v[pl.ds(l * (nbuckets + 1) + jj * 16,
                                             16)]
                merged_v[pl.ds(jj * 16, 16)] = acc
            return carry

        lax.fori_loop(0, nbuckets // 32, mbody, 0)
        pltpu.sync_copy(merged_v, out_hbm.at[wid])

    return hist


def _sless_body(keys_ref, kstar_ref, out_ref):
    i = pl.program_id(0)
    k = keys_ref[...]
    kstar = kstar_ref[0, 0]
    sel = k < kstar
    neg = k < jnp.uint32(0x80000000)
    bu = jnp.where(neg, ~k, k ^ jnp.uint32(0x80000000))
    p = lax.bitcast_convert_type(bu, jnp.float32)
    term = jnp.maximum(p, 0.0) + jnp.log1p(jnp.exp(-jnp.abs(p)))
    part = jnp.sum(jnp.where(sel, term, 0.0), axis=0)[None, :]

    @pl.when(i == 0)
    def _():
        out_ref[...] = jnp.zeros_like(out_ref)

    out_ref[...] = out_ref[...] + part


def _sless(keys2, kstar):
    return pl.pallas_call(
        _sless_body,
        grid=(ROWS // 512,),
        in_specs=[pl.BlockSpec((512, COLS), lambda i: (i, 0)),
                  pl.BlockSpec(memory_space=pltpu.SMEM)],
        out_specs=pl.BlockSpec((1, 128), lambda i: (0, 0)),
        out_shape=jax.ShapeDtypeStruct((1, 128), jnp.float32),
    )(keys2, kstar)


def _bucket_step(hist32, rank):
    h = jnp.sum(hist32, axis=0)
    c = jnp.cumsum(h)
    b = jnp.argmax(c > rank).astype(jnp.int32)
    below = jnp.where(b > 0, c[jnp.maximum(b - 1, 0)], 0)
    return b, below


def kernel(logit, target):
    npad = NP - logit.shape[0]
    p2 = jnp.pad(logit, ((0, npad), (0, 0))).reshape(ROWS, COLS)
    y2 = jnp.pad(target, ((0, npad), (0, 0)),
                 constant_values=-1.0).reshape(ROWS, COLS)
    keys2, stats = _stats_keys(p2, y2)
    s = stats[0]
    num_pos = s[0].astype(jnp.int32)
    num_neg = s[1].astype(jnp.int32)
    s_pos, s_negall, max_neg, max_pos = s[2], s[3], s[4], s[5]
    topm = jnp.minimum(num_pos, num_neg) - 1
    take = topm > 0
    r = jnp.maximum(topm - 1, 0)

    keys_flat = keys2.reshape(-1)
    full = lambda v: jnp.full((16,), v, jnp.uint32)
    h1 = _make_hist(21, 2048, 0xFFFFFFFE)(keys_flat, full(jnp.uint32(0)))
    b1, cb1 = _bucket_step(h1, r)
    lo1 = b1.astype(jnp.uint32) << 21
    h2 = _make_hist(10, 2048, (1 << 21) - 1)(keys_flat, full(lo1))
    b2, cb2 = _bucket_step(h2, r - cb1)
    lo2 = lo1 + (b2.astype(jnp.uint32) << 10)
    h3 = _make_hist(0, 1024, 1023)(keys_flat, full(lo2))
    lo2_eff = jnp.where(take, lo2, jnp.uint32(0))
    s12 = jnp.sum(_sless(keys2, lo2_eff.reshape(1, 1)))

    htot3 = jnp.sum(h3, axis=0)
    c3 = jnp.cumsum(htot3)
    b3 = jnp.argmax(c3 > (r - cb1 - cb2)).astype(jnp.int32)
    cb3 = jnp.where(b3 > 0, c3[jnp.maximum(b3 - 1, 0)], 0)
    kstar = lo2 + b3.astype(jnp.uint32)
    cnt_less = cb1 + cb2 + cb3

    jidx = jnp.arange(1024, dtype=jnp.uint32)
    kj = lo2 + jidx
    buj = jnp.where(kj >= jnp.uint32(0x80000000),
                    kj ^ jnp.uint32(0x80000000), ~kj)
    vj = lax.bitcast_convert_type(buj, jnp.float32)
    tj = jnp.maximum(vj, 0.0) + jnp.log1p(jnp.exp(-jnp.abs(vj)))
    selj = (jidx < b3.astype(jnp.uint32)) & (htot3 > 0)
    s3 = jnp.sum(jnp.where(selj, htot3.astype(jnp.float32) * tj, 0.0))
    s_less = s12 + s3

    bu = jnp.where(kstar >= jnp.uint32(0x80000000),
                   kstar ^ jnp.uint32(0x80000000), ~kstar)
    vstar = lax.bitcast_convert_type(bu, jnp.float32)
    tstar = jnp.maximum(vstar, 0.0) + jnp.log1p(jnp.exp(-jnp.abs(vstar)))
    s_sel = s_less + (topm - cnt_less).astype(jnp.float32) * tstar

    loss_bce = jnp.where(take, s_pos + s_sel, s_pos + s_negall)
    rank_val = jnp.maximum(0.0, 1.0 - max_neg + max_pos)
    loss_total = loss_bce + 0.1 * jnp.where(take, rank_val, 0.0)
    count = jnp.where(take, num_pos + topm, num_pos + num_neg).astype(jnp.int32)
    return loss_total, count
